# baseline (speedup 1.0000x reference)
import functools

import jax
import jax.numpy as jnp
from jax import lax
from jax.experimental import pallas as pl
from jax.experimental.pallas import tpu as pltpu
from jax.experimental.pallas import tpu_sc as plsc

NEMB = 32
N_NODES = 10000
NP = 10240
NBLK = 1024
NT = 20
NBLOCKS = 106
NSTEMT = 21
E = 160000
EP = 163840
ROWS_PW = 40
QLEN = NP * NT
QSL = QLEN // 16
QSP = QLEN
ZERO_BIN = N_NODES * NT
DUMP = 0
NUM_CONV_STEPS = 6
N_STEMS = 2000
SP = 2048
NGRAPH = 250
GP = 256
NODES_PER_GRAPH = 40
OUT_PER_STEM = 105

_mesh = plsc.VectorSubcoreMesh(core_axis_name="c", subcore_axis_name="s")


def _lrelu(t):
    return jnp.where(t > 0, t, 0.01 * t)


PSL = QLEN // 16
DSL = 1024
DSP = 16 * DSL
DUMP_D = N_NODES


def _make_sc_pass(with_deg):
    outs = [jax.ShapeDtypeStruct((2, QLEN), jnp.float32)]
    scratch = [
        pltpu.VMEM((ROWS_PW, 128), jnp.int32),
        pltpu.VMEM((ROWS_PW, 128), jnp.int32),
        pltpu.VMEM((ROWS_PW, 128), jnp.float32),
        pltpu.VMEM((QSL,), jnp.float32),
        pltpu.VMEM((PSL,), jnp.float32),
        pltpu.VMEM_SHARED((QSP,), jnp.float32),
        pltpu.VMEM_SHARED((QLEN,), jnp.float32),
        pltpu.SemaphoreType.DMA,
        pltpu.SemaphoreType.DMA,
    ]
    if with_deg:
        outs.append(jax.ShapeDtypeStruct((2, 16, DSL), jnp.float32))
        scratch += [
            pltpu.VMEM((ROWS_PW, 128), jnp.int32),
            pltpu.VMEM((128,), jnp.float32),
            pltpu.VMEM_SHARED((DSP,), jnp.float32),
            pltpu.SemaphoreType.DMA,
        ]

    def body(*refs):
        if with_deg:
            (p_hbm, gidx_hbm, sidx_hbm, dstidx_hbm, qpart, degpart,
             gidx_v, sidx_v, vals_v, zb_v, pb_v, q_sh, p_sh, sem_g, sem_s,
             dstidx_v, ones_v, d_sh, sem_d) = refs
        else:
            (p_hbm, gidx_hbm, sidx_hbm, qpart,
             gidx_v, sidx_v, vals_v, zb_v, pb_v, q_sh, p_sh,
             sem_g, sem_s) = refs
        c = lax.axis_index("c")
        s = lax.axis_index("s")
        wid = s * 2 + c

        h_g = pltpu.async_copy(gidx_hbm.at[pl.ds(wid * ROWS_PW, ROWS_PW)],
                               gidx_v, sem_g)
        h_s = pltpu.async_copy(sidx_hbm.at[pl.ds(wid * ROWS_PW, ROWS_PW)],
                               sidx_v, sem_g)
        h_p = pltpu.async_copy(p_hbm.at[pl.ds(s * PSL, PSL)], pb_v, sem_g)
        if with_deg:
            h_d = pltpu.async_copy(
                dstidx_hbm.at[pl.ds(wid * ROWS_PW, ROWS_PW)], dstidx_v, sem_g)
            for u in range(8):
                ones_v[pl.ds(u * 16, 16)] = jnp.ones((16,), jnp.float32)

        def _zz(i, carry):
            for u in range(8):
                zb_v[pl.ds((i * 8 + u) * 16, 16)] = jnp.zeros((16,),
                                                              jnp.float32)
            return carry

        lax.fori_loop(0, QSL // 128, _zz, 0)
        pltpu.sync_copy(zb_v, q_sh.at[pl.ds(s * QSL, QSL)])
        if with_deg:
            pltpu.sync_copy(zb_v.at[pl.ds(0, DSL)],
                            d_sh.at[pl.ds(s * DSL, DSL)])
        h_p.wait()
        pltpu.sync_copy(pb_v, p_sh.at[pl.ds(s * PSL, PSL)])
        h_g.wait()
        h_s.wait()
        if with_deg:
            h_d.wait()
        plsc.subcore_barrier()

        def _grp(g, carry):
            handles = []
            for j in range(8):
                r = g * 8 + j
                handles.append(
                    pltpu.async_copy(p_sh.at[gidx_v.at[r]], vals_v.at[r],
                                     sem_g))
            for h in handles:
                h.wait()
            for j in range(8):
                r = g * 8 + j
                pltpu.async_copy(vals_v.at[r], q_sh.at[sidx_v.at[r]], sem_s,
                                 add=True)
                if with_deg:
                    pltpu.async_copy(ones_v, d_sh.at[dstidx_v.at[r]], sem_d,
                                     add=True)
            return carry

        lax.fori_loop(0, ROWS_PW // 8, _grp, 0)

        def _drain(i, carry):
            pltpu.make_async_copy(vals_v.at[i], q_sh.at[sidx_v.at[i]],
                                  sem_s).wait()
            if with_deg:
                pltpu.make_async_copy(ones_v, d_sh.at[dstidx_v.at[i]],
                                      sem_d).wait()
            return carry

        lax.fori_loop(0, ROWS_PW, _drain, 0)
        plsc.subcore_barrier()

        pltpu.sync_copy(q_sh.at[pl.ds(s * QSL, QSL)],
                        qpart.at[c, pl.ds(s * QSL, QSL)])
        if with_deg:
            pltpu.sync_copy(d_sh.at[pl.ds(s * DSL, DSL)], degpart.at[c, s])

    return functools.partial(
        pl.kernel, mesh=_mesh,
        out_type=outs if with_deg else outs[0],
        scratch_types=scratch)(body)


_sc_pass = _make_sc_pass(False)
_sc_pass_deg = _make_sc_pass(True)


def _pro_body(x_ref, be_ref, w1_ref, b1_ref, w2_ref, b2_ref, bond_ref,
              out_ref, p_ref):
    xo = x_ref[...]
    oh = (lax.broadcasted_iota(jnp.int32, (NBLK, NBLOCKS), 1) == xo)
    xe = jnp.dot(oh.astype(jnp.float32), be_ref[...],
                 preferred_element_type=jnp.float32, precision=lax.Precision.HIGHEST)
    h1 = _lrelu(jnp.dot(xe, w1_ref[...], preferred_element_type=jnp.float32, precision=lax.Precision.HIGHEST)
                + b1_ref[...])
    o0 = jnp.dot(h1, w2_ref[...], preferred_element_type=jnp.float32, precision=lax.Precision.HIGHEST) + b2_ref[...]
    out_ref[...] = o0
    p = lax.dot_general(o0, bond_ref[...], (((1,), (1,)), ((), ())),
                        preferred_element_type=jnp.float32, precision=lax.Precision.HIGHEST)
    rows = (pl.program_id(0) * NBLK
            + lax.broadcasted_iota(jnp.int32, (NBLK, NT), 0))
    p_ref[...] = jnp.where(rows < N_NODES, p, 0.0)


_tc_pro = pl.pallas_call(
    _pro_body,
    grid=(NP // NBLK,),
    in_specs=[
        pl.BlockSpec((NBLK, 1), lambda i: (i, 0)),
        pl.BlockSpec((NBLOCKS, NEMB), lambda i: (0, 0)),
        pl.BlockSpec((NEMB, NEMB), lambda i: (0, 0)),
        pl.BlockSpec((1, NEMB), lambda i: (0, 0)),
        pl.BlockSpec((NEMB, NEMB), lambda i: (0, 0)),
        pl.BlockSpec((1, NEMB), lambda i: (0, 0)),
        pl.BlockSpec((NT, NEMB), lambda i: (0, 0)),
    ],
    out_specs=[
        pl.BlockSpec((NBLK, NEMB), lambda i: (i, 0)),
        pl.BlockSpec((NBLK, NT), lambda i: (i, 0)),
    ],
    out_shape=[
        jax.ShapeDtypeStruct((NP, NEMB), jnp.float32),
        jax.ShapeDtypeStruct((NP, NT), jnp.float32),
    ],
)


def _step_body(out_in, q_ref, deg_ref, bond_ref, root_ref, cb_ref,
               wih_ref, whh_ref, bih_ref, bhh_ref, out_new, p_ref):
    h = out_in[...]
    q = q_ref[0] + q_ref[1]
    agg = jnp.dot(q, bond_ref[...],
                  preferred_element_type=jnp.float32, precision=lax.Precision.HIGHEST) / jnp.maximum(deg_ref[...], 1.0)
    conv = agg + jnp.dot(h, root_ref[...],
                         preferred_element_type=jnp.float32, precision=lax.Precision.HIGHEST) + cb_ref[...]
    m = _lrelu(conv)
    gi = lax.dot_general(m, wih_ref[...], (((1,), (1,)), ((), ())),
                         preferred_element_type=jnp.float32, precision=lax.Precision.HIGHEST) + bih_ref[...]
    gh = lax.dot_general(h, whh_ref[...], (((1,), (1,)), ((), ())),
                         preferred_element_type=jnp.float32, precision=lax.Precision.HIGHEST) + bhh_ref[...]
    r = jax.nn.sigmoid(gi[:, :NEMB] + gh[:, :NEMB])
    z = jax.nn.sigmoid(gi[:, NEMB:2 * NEMB] + gh[:, NEMB:2 * NEMB])
    n = jnp.tanh(gi[:, 2 * NEMB:] + r * gh[:, 2 * NEMB:])
    hn = (1.0 - z) * n + z * h
    out_new[...] = hn
    p = lax.dot_general(hn, bond_ref[...], (((1,), (1,)), ((), ())),
                        preferred_element_type=jnp.float32, precision=lax.Precision.HIGHEST)
    rows = (pl.program_id(0) * NBLK
            + lax.broadcasted_iota(jnp.int32, (NBLK, NT), 0))
    p_ref[...] = jnp.where(rows < N_NODES, p, 0.0)


_tc_step = pl.pallas_call(
    _step_body,
    grid=(NP // NBLK,),
    in_specs=[
        pl.BlockSpec((NBLK, NEMB), lambda i: (i, 0)),
        pl.BlockSpec((2, NBLK, NT), lambda i: (0, i, 0)),
        pl.BlockSpec((NBLK, 1), lambda i: (i, 0)),
        pl.BlockSpec((NT, NEMB), lambda i: (0, 0)),
        pl.BlockSpec((NEMB, NEMB), lambda i: (0, 0)),
        pl.BlockSpec((1, NEMB), lambda i: (0, 0)),
        pl.BlockSpec((3 * NEMB, NEMB), lambda i: (0, 0)),
        pl.BlockSpec((3 * NEMB, NEMB), lambda i: (0, 0)),
        pl.BlockSpec((1, 3 * NEMB), lambda i: (0, 0)),
        pl.BlockSpec((1, 3 * NEMB), lambda i: (0, 0)),
    ],
    out_specs=[
        pl.BlockSpec((NBLK, NEMB), lambda i: (i, 0)),
        pl.BlockSpec((NBLK, NT), lambda i: (i, 0)),
    ],
    out_shape=[
        jax.ShapeDtypeStruct((NP, NEMB), jnp.float32),
        jax.ShapeDtypeStruct((NP, NT), jnp.float32),
    ],
)


def _stem_body(out_ref, sid_ref, st_ref, semb_ref, w1a_ref, w1b_ref, b1_ref,
               w2_ref, b2_ref, w3_ref, b3_ref, preds_ref, acc_ref):
    j = pl.program_id(1)
    sid = sid_ref[...]
    oh = (sid == j * NBLK
          + lax.broadcasted_iota(jnp.int32, (NBLK, NBLK), 1))
    part = jnp.dot(oh.astype(jnp.float32), out_ref[...],
                   preferred_element_type=jnp.float32,
                   precision=lax.Precision.HIGHEST)

    @pl.when(j == 0)
    def _():
        acc_ref[...] = part

    @pl.when(j > 0)
    def _():
        acc_ref[...] = acc_ref[...] + part

    @pl.when(j == NP // NBLK - 1)
    def _():
        so = st_ref[...]
        ohs = (lax.broadcasted_iota(jnp.int32, (NBLK, NSTEMT), 1) == so)
        semb = jnp.dot(ohs.astype(jnp.float32), semb_ref[...],
                       preferred_element_type=jnp.float32, precision=lax.Precision.HIGHEST)
        h1 = _lrelu(jnp.dot(acc_ref[...], w1a_ref[...],
                            preferred_element_type=jnp.float32, precision=lax.Precision.HIGHEST)
                    + jnp.dot(semb, w1b_ref[...],
                              preferred_element_type=jnp.float32, precision=lax.Precision.HIGHEST) + b1_ref[...])
        h2 = _lrelu(jnp.dot(h1, w2_ref[...],
                            preferred_element_type=jnp.float32, precision=lax.Precision.HIGHEST) + b2_ref[...])
        preds_ref[...] = jnp.dot(h2, w3_ref[...],
                                 preferred_element_type=jnp.float32, precision=lax.Precision.HIGHEST) + b3_ref[...]


_tc_stem = pl.pallas_call(
    _stem_body,
    grid=(SP // NBLK, NP // NBLK),
    in_specs=[
        pl.BlockSpec((NBLK, NEMB), lambda i, j: (j, 0)),
        pl.BlockSpec((NBLK, 1), lambda i, j: (i, 0)),
        pl.BlockSpec((NBLK, 1), lambda i, j: (i, 0)),
        pl.BlockSpec((NSTEMT, NEMB), lambda i, j: (0, 0)),
        pl.BlockSpec((NEMB, NEMB), lambda i, j: (0, 0)),
        pl.BlockSpec((NEMB, NEMB), lambda i, j: (0, 0)),
        pl.BlockSpec((1, NEMB), lambda i, j: (0, 0)),
        pl.BlockSpec((NEMB, NEMB), lambda i, j: (0, 0)),
        pl.BlockSpec((1, NEMB), lambda i, j: (0, 0)),
        pl.BlockSpec((NEMB, OUT_PER_STEM), lambda i, j: (0, 0)),
        pl.BlockSpec((1, OUT_PER_STEM), lambda i, j: (0, 0)),
    ],
    out_specs=pl.BlockSpec((NBLK, OUT_PER_STEM), lambda i, j: (i, 0)),
    out_shape=jax.ShapeDtypeStruct((SP, OUT_PER_STEM), jnp.float32),
    scratch_shapes=[pltpu.VMEM((NBLK, NEMB), jnp.float32)],
)


def _mol_body(og_ref, w1_ref, b1_ref, w2_ref, b2_ref, preds_ref):
    blk = og_ref[...].reshape(64, NODES_PER_GRAPH, NEMB)
    gmean = jnp.mean(blk, axis=1)
    h1 = _lrelu(jnp.dot(gmean, w1_ref[...],
                        preferred_element_type=jnp.float32, precision=lax.Precision.HIGHEST) + b1_ref[...])
    preds_ref[...] = jnp.dot(h1, w2_ref[...],
                             preferred_element_type=jnp.float32, precision=lax.Precision.HIGHEST) + b2_ref[...]


_tc_mol = pl.pallas_call(
    _mol_body,
    grid=(4,),
    in_specs=[
        pl.BlockSpec((64 * NODES_PER_GRAPH, NEMB), lambda i: (i, 0)),
        pl.BlockSpec((NEMB, NEMB), lambda i: (0, 0)),
        pl.BlockSpec((1, NEMB), lambda i: (0, 0)),
        pl.BlockSpec((NEMB, 1), lambda i: (0, 0)),
        pl.BlockSpec((1, 1), lambda i: (0, 0)),
    ],
    out_specs=pl.BlockSpec((64, 1), lambda i: (i, 0)),
    out_shape=jax.ShapeDtypeStruct((GP, 1), jnp.float32),
)


def kernel(x, edge_index, edge_attr, stemtypes, stems, batch, stems_batch,
           x_slices, blockemb, stememb, bondemb, b2e_W1, b2e_b1, b2e_W2,
           b2e_b2, conv_root, conv_bias, gru_Wih, gru_Whh, gru_bih, gru_bhh,
           s2p_W1, s2p_b1, s2p_W2, s2p_b2, s2p_W3, s2p_b3,
           g2p_W1, g2p_b1, g2p_W2, g2p_b2):
    i32 = jnp.int32
    f32 = jnp.float32

    src = edge_index[0].astype(i32)
    dst = edge_index[1].astype(i32)
    t0 = edge_attr[:, 0].astype(i32)
    t1 = edge_attr[:, 1].astype(i32)
    gidx = jnp.concatenate([src * NT + t0, jnp.full((EP - E,), ZERO_BIN, i32)])
    sidx = jnp.concatenate([dst * NT + t1, jnp.full((EP - E,), DUMP, i32)])
    gidx = gidx.reshape(EP // 128, 128)
    sidx = sidx.reshape(EP // 128, 128)
    dstidx = jnp.concatenate([dst, jnp.full((EP - E,), DUMP_D, i32)])
    dstidx = dstidx.reshape(EP // 128, 128)

    stem_idx = stems_batch.astype(i32) * NODES_PER_GRAPH + stems[:, 0].astype(i32)
    stem_idx = jnp.concatenate([stem_idx,
                                jnp.zeros((SP - N_STEMS,), i32)]).reshape(SP, 1)

    x_pad = jnp.concatenate([x.astype(i32), jnp.zeros((NP - N_NODES,), i32)])
    x_pad = x_pad.reshape(NP, 1)
    st_pad = jnp.concatenate([stemtypes.astype(i32),
                              jnp.zeros((SP - N_STEMS,), i32)]).reshape(SP, 1)

    b2e_b1r = b2e_b1.reshape(1, NEMB)
    b2e_b2r = b2e_b2.reshape(1, NEMB)
    cb = conv_bias.reshape(1, NEMB)
    bih = gru_bih.reshape(1, 3 * NEMB)
    bhh = gru_bhh.reshape(1, 3 * NEMB)
    w1a = s2p_W1[:NEMB]
    w1b = s2p_W1[NEMB:]
    s2p_b1r = s2p_b1.reshape(1, NEMB)
    s2p_b2r = s2p_b2.reshape(1, NEMB)
    s2p_b3r = s2p_b3.reshape(1, OUT_PER_STEM)
    g2p_b1r = g2p_b1.reshape(1, NEMB)
    g2p_b2r = g2p_b2.reshape(1, 1)

    out, P = _tc_pro(x_pad, blockemb, b2e_W1, b2e_b1r,
                     b2e_W2, b2e_b2r, bondemb)

    qpart, degpart = _sc_pass_deg(P.reshape(-1), gidx, sidx, dstidx)
    d2 = degpart.reshape(2, DSP)[:, :NP]
    deg = (d2[0] + d2[1]).reshape(NP, 1)
    for step in range(NUM_CONV_STEPS):
        if step > 0:
            qpart = _sc_pass(P.reshape(-1), gidx, sidx)
        q3 = qpart.reshape(2, NP, NT)
        out, P = _tc_step(out, q3, deg, bondemb, conv_root, cb,
                          gru_Wih, gru_Whh, bih, bhh)

    stem_preds = _tc_stem(out, stem_idx, st_pad, stememb, w1a, w1b, s2p_b1r,
                          s2p_W2, s2p_b2r, s2p_W3, s2p_b3r)[:N_STEMS]

    mol_preds = _tc_mol(out, g2p_W1, g2p_b1r, g2p_W2, g2p_b2r)[:NGRAPH]

    return stem_preds, mol_preds


# speedup vs baseline: 1.1039x; 1.1039x over previous
import functools

import jax
import jax.numpy as jnp
from jax import lax
from jax.experimental import pallas as pl
from jax.experimental.pallas import tpu as pltpu
from jax.experimental.pallas import tpu_sc as plsc

NEMB = 32
N_NODES = 10000
NP = 10240
NBLK = 1024
NT = 20
NBLOCKS = 106
NSTEMT = 21
E = 160000
EP = 163840
ROWS_PW = 40
QLEN = NP * NT
QSL = QLEN // 16
QSP = QLEN
ZERO_BIN = N_NODES * NT
DUMP = 0
NUM_CONV_STEPS = 6
N_STEMS = 2000
SP = 2048
NGRAPH = 250
GP = 256
NODES_PER_GRAPH = 40
OUT_PER_STEM = 105

_mesh = plsc.VectorSubcoreMesh(core_axis_name="c", subcore_axis_name="s")


def _lrelu(t):
    return jnp.where(t > 0, t, 0.01 * t)


PSL = QLEN // 16
DSL = 1024
DSP = 16 * DSL
DUMP_D = N_NODES


def _make_sc_pass(with_deg):
    outs = [jax.ShapeDtypeStruct((2, QLEN), jnp.float32)]
    scratch = [
        pltpu.VMEM((ROWS_PW, 128), jnp.int32),
        pltpu.VMEM((ROWS_PW, 128), jnp.int32),
        pltpu.VMEM((ROWS_PW, 128), jnp.float32),
        pltpu.VMEM((QSL,), jnp.float32),
        pltpu.VMEM((PSL,), jnp.float32),
        pltpu.VMEM_SHARED((QSP,), jnp.float32),
        pltpu.VMEM_SHARED((QLEN,), jnp.float32),
        pltpu.SemaphoreType.DMA,
        pltpu.SemaphoreType.DMA,
    ]
    if with_deg:
        outs.append(jax.ShapeDtypeStruct((2, 16, DSL), jnp.float32))
        scratch += [
            pltpu.VMEM((ROWS_PW, 128), jnp.int32),
            pltpu.VMEM((128,), jnp.float32),
            pltpu.VMEM_SHARED((DSP,), jnp.float32),
            pltpu.SemaphoreType.DMA,
        ]

    def body(*refs):
        if with_deg:
            (p_hbm, gidx_hbm, sidx_hbm, dstidx_hbm, qpart, degpart,
             gidx_v, sidx_v, vals_v, zb_v, pb_v, q_sh, p_sh, sem_g, sem_s,
             dstidx_v, ones_v, d_sh, sem_d) = refs
        else:
            (p_hbm, gidx_hbm, sidx_hbm, qpart,
             gidx_v, sidx_v, vals_v, zb_v, pb_v, q_sh, p_sh,
             sem_g, sem_s) = refs
        c = lax.axis_index("c")
        s = lax.axis_index("s")
        wid = s * 2 + c

        h_g = pltpu.async_copy(gidx_hbm.at[pl.ds(wid * ROWS_PW, ROWS_PW)],
                               gidx_v, sem_g)
        h_s = pltpu.async_copy(sidx_hbm.at[pl.ds(wid * ROWS_PW, ROWS_PW)],
                               sidx_v, sem_g)
        h_p = pltpu.async_copy(p_hbm.at[pl.ds(s * PSL, PSL)], pb_v, sem_g)
        if with_deg:
            h_d = pltpu.async_copy(
                dstidx_hbm.at[pl.ds(wid * ROWS_PW, ROWS_PW)], dstidx_v, sem_g)
            for u in range(8):
                ones_v[pl.ds(u * 16, 16)] = jnp.ones((16,), jnp.float32)

        def _zz(i, carry):
            for u in range(8):
                zb_v[pl.ds((i * 8 + u) * 16, 16)] = jnp.zeros((16,),
                                                              jnp.float32)
            return carry

        lax.fori_loop(0, QSL // 128, _zz, 0)
        pltpu.sync_copy(zb_v, q_sh.at[pl.ds(s * QSL, QSL)])
        if with_deg:
            pltpu.sync_copy(zb_v.at[pl.ds(0, DSL)],
                            d_sh.at[pl.ds(s * DSL, DSL)])
        h_p.wait()
        pltpu.sync_copy(pb_v, p_sh.at[pl.ds(s * PSL, PSL)])
        h_g.wait()
        h_s.wait()
        if with_deg:
            h_d.wait()
        plsc.subcore_barrier()

        def _grp(g, carry):
            handles = []
            for j in range(8):
                r = g * 8 + j
                handles.append(
                    pltpu.async_copy(p_sh.at[gidx_v.at[r]], vals_v.at[r],
                                     sem_g))
            for h in handles:
                h.wait()
            for j in range(8):
                r = g * 8 + j
                pltpu.async_copy(vals_v.at[r], q_sh.at[sidx_v.at[r]], sem_s,
                                 add=True)
                if with_deg:
                    pltpu.async_copy(ones_v, d_sh.at[dstidx_v.at[r]], sem_d,
                                     add=True)
            return carry

        lax.fori_loop(0, ROWS_PW // 8, _grp, 0)

        def _drain(i, carry):
            pltpu.make_async_copy(vals_v.at[i], q_sh.at[sidx_v.at[i]],
                                  sem_s).wait()
            if with_deg:
                pltpu.make_async_copy(ones_v, d_sh.at[dstidx_v.at[i]],
                                      sem_d).wait()
            return carry

        lax.fori_loop(0, ROWS_PW, _drain, 0)
        plsc.subcore_barrier()

        pltpu.sync_copy(q_sh.at[pl.ds(s * QSL, QSL)],
                        qpart.at[c, pl.ds(s * QSL, QSL)])
        if with_deg:
            pltpu.sync_copy(d_sh.at[pl.ds(s * DSL, DSL)], degpart.at[c, s])

    return functools.partial(
        pl.kernel, mesh=_mesh,
        out_type=outs if with_deg else outs[0],
        scratch_types=scratch)(body)


_sc_pass = _make_sc_pass(False)
_sc_pass_deg = _make_sc_pass(True)


def _pro_body(x_ref, be_ref, w1_ref, b1_ref, w2_ref, b2_ref, bond_ref,
              out_ref, p_ref):
    xo = x_ref[...]
    oh = (lax.broadcasted_iota(jnp.int32, (NBLK, NBLOCKS), 1) == xo)
    xe = jnp.dot(oh.astype(jnp.float32), be_ref[...],
                 preferred_element_type=jnp.float32, precision=lax.Precision.HIGHEST)
    h1 = _lrelu(jnp.dot(xe, w1_ref[...], preferred_element_type=jnp.float32, precision=lax.Precision.HIGHEST)
                + b1_ref[...])
    o0 = jnp.dot(h1, w2_ref[...], preferred_element_type=jnp.float32, precision=lax.Precision.HIGHEST) + b2_ref[...]
    out_ref[...] = o0
    p = lax.dot_general(o0, bond_ref[...], (((1,), (1,)), ((), ())),
                        preferred_element_type=jnp.float32, precision=lax.Precision.HIGHEST)
    rows = (pl.program_id(0) * NBLK
            + lax.broadcasted_iota(jnp.int32, (NBLK, NT), 0))
    p_ref[...] = jnp.where(rows < N_NODES, p, 0.0)


_tc_pro = pl.pallas_call(
    _pro_body,
    grid=(NP // NBLK,),
    in_specs=[
        pl.BlockSpec((NBLK, 1), lambda i: (i, 0)),
        pl.BlockSpec((NBLOCKS, NEMB), lambda i: (0, 0)),
        pl.BlockSpec((NEMB, NEMB), lambda i: (0, 0)),
        pl.BlockSpec((1, NEMB), lambda i: (0, 0)),
        pl.BlockSpec((NEMB, NEMB), lambda i: (0, 0)),
        pl.BlockSpec((1, NEMB), lambda i: (0, 0)),
        pl.BlockSpec((NT, NEMB), lambda i: (0, 0)),
    ],
    out_specs=[
        pl.BlockSpec((NBLK, NEMB), lambda i: (i, 0)),
        pl.BlockSpec((NBLK, NT), lambda i: (i, 0)),
    ],
    out_shape=[
        jax.ShapeDtypeStruct((NP, NEMB), jnp.float32),
        jax.ShapeDtypeStruct((NP, NT), jnp.float32),
    ],
)


def _step_body(out_in, q_ref, deg_ref, bond_ref, root_ref, cb_ref,
               wih_ref, whh_ref, bih_ref, bhh_ref, out_new, p_ref):
    h = out_in[...]
    q = q_ref[0] + q_ref[1]
    agg = jnp.dot(q, bond_ref[...],
                  preferred_element_type=jnp.float32, precision=lax.Precision.HIGHEST) / jnp.maximum(deg_ref[...], 1.0)
    conv = agg + jnp.dot(h, root_ref[...],
                         preferred_element_type=jnp.float32, precision=lax.Precision.HIGHEST) + cb_ref[...]
    m = _lrelu(conv)
    gi = lax.dot_general(m, wih_ref[...], (((1,), (1,)), ((), ())),
                         preferred_element_type=jnp.float32, precision=lax.Precision.HIGHEST) + bih_ref[...]
    gh = lax.dot_general(h, whh_ref[...], (((1,), (1,)), ((), ())),
                         preferred_element_type=jnp.float32, precision=lax.Precision.HIGHEST) + bhh_ref[...]
    r = jax.nn.sigmoid(gi[:, :NEMB] + gh[:, :NEMB])
    z = jax.nn.sigmoid(gi[:, NEMB:2 * NEMB] + gh[:, NEMB:2 * NEMB])
    n = jnp.tanh(gi[:, 2 * NEMB:] + r * gh[:, 2 * NEMB:])
    hn = (1.0 - z) * n + z * h
    out_new[...] = hn
    p = lax.dot_general(hn, bond_ref[...], (((1,), (1,)), ((), ())),
                        preferred_element_type=jnp.float32, precision=lax.Precision.HIGHEST)
    rows = (pl.program_id(0) * NBLK
            + lax.broadcasted_iota(jnp.int32, (NBLK, NT), 0))
    p_ref[...] = jnp.where(rows < N_NODES, p, 0.0)


_tc_step = pl.pallas_call(
    _step_body,
    grid=(NP // NBLK,),
    in_specs=[
        pl.BlockSpec((NBLK, NEMB), lambda i: (i, 0)),
        pl.BlockSpec((2, NBLK, NT), lambda i: (0, i, 0)),
        pl.BlockSpec((NBLK, 1), lambda i: (i, 0)),
        pl.BlockSpec((NT, NEMB), lambda i: (0, 0)),
        pl.BlockSpec((NEMB, NEMB), lambda i: (0, 0)),
        pl.BlockSpec((1, NEMB), lambda i: (0, 0)),
        pl.BlockSpec((3 * NEMB, NEMB), lambda i: (0, 0)),
        pl.BlockSpec((3 * NEMB, NEMB), lambda i: (0, 0)),
        pl.BlockSpec((1, 3 * NEMB), lambda i: (0, 0)),
        pl.BlockSpec((1, 3 * NEMB), lambda i: (0, 0)),
    ],
    out_specs=[
        pl.BlockSpec((NBLK, NEMB), lambda i: (i, 0)),
        pl.BlockSpec((NBLK, NT), lambda i: (i, 0)),
    ],
    out_shape=[
        jax.ShapeDtypeStruct((NP, NEMB), jnp.float32),
        jax.ShapeDtypeStruct((NP, NT), jnp.float32),
    ],
)


SROWS = SP * NEMB // 128
SROWS_PW = SROWS // 32


@functools.partial(
    pl.kernel,
    mesh=_mesh,
    out_type=jax.ShapeDtypeStruct((SROWS, 128), jnp.float32),
    scratch_types=[
        pltpu.VMEM((SROWS_PW, 128), jnp.int32),
        pltpu.VMEM((SROWS_PW, 128), jnp.float32),
        pltpu.SemaphoreType.DMA,
    ],
)
def _sc_flatgather(tab_hbm, idx_hbm, vals_hbm, idx_v, vals_v, sem):
    c = lax.axis_index("c")
    s = lax.axis_index("s")
    wid = s * 2 + c
    pltpu.sync_copy(idx_hbm.at[pl.ds(wid * SROWS_PW, SROWS_PW)], idx_v)
    handles = [
        pltpu.async_copy(tab_hbm.at[idx_v.at[r]], vals_v.at[r], sem)
        for r in range(SROWS_PW)
    ]
    for h in handles:
        h.wait()
    pltpu.sync_copy(vals_v, vals_hbm.at[pl.ds(wid * SROWS_PW, SROWS_PW)])


def _stem_body(rows_ref, st_ref, semb_ref, w1a_ref, w1b_ref, b1_ref,
               w2_ref, b2_ref, w3_ref, b3_ref, preds_ref):
    so = st_ref[...]
    oh = (lax.broadcasted_iota(jnp.int32, (NBLK, NSTEMT), 1) == so)
    semb = jnp.dot(oh.astype(jnp.float32), semb_ref[...],
                   preferred_element_type=jnp.float32, precision=lax.Precision.HIGHEST)
    h1 = _lrelu(jnp.dot(rows_ref[...], w1a_ref[...],
                        preferred_element_type=jnp.float32, precision=lax.Precision.HIGHEST)
                + jnp.dot(semb, w1b_ref[...],
                          preferred_element_type=jnp.float32, precision=lax.Precision.HIGHEST) + b1_ref[...])
    h2 = _lrelu(jnp.dot(h1, w2_ref[...],
                        preferred_element_type=jnp.float32, precision=lax.Precision.HIGHEST) + b2_ref[...])
    preds_ref[...] = jnp.dot(h2, w3_ref[...],
                             preferred_element_type=jnp.float32, precision=lax.Precision.HIGHEST) + b3_ref[...]


_tc_stem = pl.pallas_call(
    _stem_body,
    grid=(SP // NBLK,),
    in_specs=[
        pl.BlockSpec((NBLK, NEMB), lambda i: (i, 0)),
        pl.BlockSpec((NBLK, 1), lambda i: (i, 0)),
        pl.BlockSpec((NSTEMT, NEMB), lambda i: (0, 0)),
        pl.BlockSpec((NEMB, NEMB), lambda i: (0, 0)),
        pl.BlockSpec((NEMB, NEMB), lambda i: (0, 0)),
        pl.BlockSpec((1, NEMB), lambda i: (0, 0)),
        pl.BlockSpec((NEMB, NEMB), lambda i: (0, 0)),
        pl.BlockSpec((1, NEMB), lambda i: (0, 0)),
        pl.BlockSpec((NEMB, OUT_PER_STEM), lambda i: (0, 0)),
        pl.BlockSpec((1, OUT_PER_STEM), lambda i: (0, 0)),
    ],
    out_specs=pl.BlockSpec((NBLK, OUT_PER_STEM), lambda i: (i, 0)),
    out_shape=jax.ShapeDtypeStruct((SP, OUT_PER_STEM), jnp.float32),
)


def _mol_body(og_ref, w1_ref, b1_ref, w2_ref, b2_ref, preds_ref):
    blk = og_ref[...].reshape(64, NODES_PER_GRAPH, NEMB)
    gmean = jnp.mean(blk, axis=1)
    h1 = _lrelu(jnp.dot(gmean, w1_ref[...],
                        preferred_element_type=jnp.float32, precision=lax.Precision.HIGHEST) + b1_ref[...])
    preds_ref[...] = jnp.dot(h1, w2_ref[...],
                             preferred_element_type=jnp.float32, precision=lax.Precision.HIGHEST) + b2_ref[...]


_tc_mol = pl.pallas_call(
    _mol_body,
    grid=(4,),
    in_specs=[
        pl.BlockSpec((64 * NODES_PER_GRAPH, NEMB), lambda i: (i, 0)),
        pl.BlockSpec((NEMB, NEMB), lambda i: (0, 0)),
        pl.BlockSpec((1, NEMB), lambda i: (0, 0)),
        pl.BlockSpec((NEMB, 1), lambda i: (0, 0)),
        pl.BlockSpec((1, 1), lambda i: (0, 0)),
    ],
    out_specs=pl.BlockSpec((64, 1), lambda i: (i, 0)),
    out_shape=jax.ShapeDtypeStruct((GP, 1), jnp.float32),
)


def kernel(x, edge_index, edge_attr, stemtypes, stems, batch, stems_batch,
           x_slices, blockemb, stememb, bondemb, b2e_W1, b2e_b1, b2e_W2,
           b2e_b2, conv_root, conv_bias, gru_Wih, gru_Whh, gru_bih, gru_bhh,
           s2p_W1, s2p_b1, s2p_W2, s2p_b2, s2p_W3, s2p_b3,
           g2p_W1, g2p_b1, g2p_W2, g2p_b2):
    i32 = jnp.int32
    f32 = jnp.float32

    src = edge_index[0].astype(i32)
    dst = edge_index[1].astype(i32)
    t0 = edge_attr[:, 0].astype(i32)
    t1 = edge_attr[:, 1].astype(i32)
    gidx = jnp.concatenate([src * NT + t0, jnp.full((EP - E,), ZERO_BIN, i32)])
    sidx = jnp.concatenate([dst * NT + t1, jnp.full((EP - E,), DUMP, i32)])
    gidx = gidx.reshape(EP // 128, 128)
    sidx = sidx.reshape(EP // 128, 128)
    dstidx = jnp.concatenate([dst, jnp.full((EP - E,), DUMP_D, i32)])
    dstidx = dstidx.reshape(EP // 128, 128)

    stem_idx = stems_batch.astype(i32) * NODES_PER_GRAPH + stems[:, 0].astype(i32)
    stem_idx = jnp.concatenate([stem_idx, jnp.zeros((SP - N_STEMS,), i32)])
    stem_fidx = (stem_idx[:, None] * NEMB
                 + jnp.arange(NEMB, dtype=i32)[None, :]).reshape(SROWS, 128)

    x_pad = jnp.concatenate([x.astype(i32), jnp.zeros((NP - N_NODES,), i32)])
    x_pad = x_pad.reshape(NP, 1)
    st_pad = jnp.concatenate([stemtypes.astype(i32),
                              jnp.zeros((SP - N_STEMS,), i32)]).reshape(SP, 1)

    b2e_b1r = b2e_b1.reshape(1, NEMB)
    b2e_b2r = b2e_b2.reshape(1, NEMB)
    cb = conv_bias.reshape(1, NEMB)
    bih = gru_bih.reshape(1, 3 * NEMB)
    bhh = gru_bhh.reshape(1, 3 * NEMB)
    w1a = s2p_W1[:NEMB]
    w1b = s2p_W1[NEMB:]
    s2p_b1r = s2p_b1.reshape(1, NEMB)
    s2p_b2r = s2p_b2.reshape(1, NEMB)
    s2p_b3r = s2p_b3.reshape(1, OUT_PER_STEM)
    g2p_b1r = g2p_b1.reshape(1, NEMB)
    g2p_b2r = g2p_b2.reshape(1, 1)

    out, P = _tc_pro(x_pad, blockemb, b2e_W1, b2e_b1r,
                     b2e_W2, b2e_b2r, bondemb)

    qpart, degpart = _sc_pass_deg(P.reshape(-1), gidx, sidx, dstidx)
    d2 = degpart.reshape(2, DSP)[:, :NP]
    deg = (d2[0] + d2[1]).reshape(NP, 1)
    for step in range(NUM_CONV_STEPS):
        if step > 0:
            qpart = _sc_pass(P.reshape(-1), gidx, sidx)
        q3 = qpart.reshape(2, NP, NT)
        out, P = _tc_step(out, q3, deg, bondemb, conv_root, cb,
                          gru_Wih, gru_Whh, bih, bhh)

    srows = _sc_flatgather(out.reshape(-1), stem_fidx).reshape(SP, NEMB)
    stem_preds = _tc_stem(srows, st_pad, stememb, w1a, w1b, s2p_b1r,
                          s2p_W2, s2p_b2r, s2p_W3, s2p_b3r)[:N_STEMS]

    mol_preds = _tc_mol(out, g2p_W1, g2p_b1r, g2p_W2, g2p_b2r)[:NGRAPH]

    return stem_preds, mol_preds
